# baseline (speedup 1.0000x reference)
import functools

import jax
import jax.numpy as jnp
from jax import lax
from jax.experimental import pallas as pl
from jax.experimental.pallas import tpu as pltpu
from jax.experimental.pallas import tpu_sc as plsc

BATCH = 1024
INPUT_SIZE = 1024
NUM_NODES = 1024
N_IN = 6

_NC, _NS = 2, 16
_NW = _NC * _NS

_HALF = NUM_NODES
_HROWS = N_IN * _HALF
_CHUNK = _HROWS // _NW
_SUB = 48
_NSUB = _CHUNK // _SUB


def _sc_gather_half(xt, flat_idx):
    @functools.partial(
        pl.kernel,
        mesh=plsc.VectorSubcoreMesh(core_axis_name="c", subcore_axis_name="s"),
        out_type=jax.ShapeDtypeStruct((_HROWS, BATCH), jnp.float32),
        scratch_types=[
            pltpu.VMEM((_SUB,), jnp.int32),
            pltpu.VMEM((_SUB,), jnp.int32),
            pltpu.VMEM((_SUB, BATCH), jnp.float32),
            pltpu.VMEM((_SUB, BATCH), jnp.float32),
            pltpu.SemaphoreType.DMA,
            pltpu.SemaphoreType.DMA,
            pltpu.SemaphoreType.DMA,
        ],
    )
    def body(xt_hbm, idx_hbm, out_hbm, idx0, idx1, rows0, rows1, gsem,
             wsem0, wsem1):
        wid = lax.axis_index("s") * _NC + lax.axis_index("c")
        off = wid * _CHUNK
        idx_bufs = (idx0, idx1)
        row_bufs = (rows0, rows1)
        wsems = (wsem0, wsem1)
        wbs = [None, None]
        for t in range(_NSUB):
            p = t % 2
            if wbs[p] is not None:
                wbs[p].wait()
            o = off + t * _SUB
            pltpu.sync_copy(idx_hbm.at[pl.ds(o, _SUB)], idx_bufs[p])
            pltpu.async_copy(xt_hbm.at[idx_bufs[p]], row_bufs[p], gsem).wait()
            wbs[p] = pltpu.async_copy(row_bufs[p], out_hbm.at[pl.ds(o, _SUB)],
                                      wsems[p])
        wbs[0].wait()
        wbs[1].wait()

    return body(xt, flat_idx)


_NT = 128


def _fold_body(luts_ref, g_ref, out_ref):
    g = [g_ref[i] for i in range(N_IN)]

    def t(level, j):
        if level == 1:
            a = luts_ref[:, 2 * j:2 * j + 1]
            b = luts_ref[:, 2 * j + 1:2 * j + 2]
            return a + g[0] * (b - a)
        a = t(level - 1, 2 * j)
        b = t(level - 1, 2 * j + 1)
        return a + g[level - 1] * (b - a)

    out_ref[...] = t(N_IN, 0).T


def _tc_fold_half(luts_h, g3, col0, prev=None):
    steps = _HALF // _NT
    base = col0 // _NT
    in_specs = [
        pl.BlockSpec((_NT, 2 ** N_IN), lambda j: (j, 0)),
        pl.BlockSpec((N_IN, _NT, BATCH), lambda j: (0, j, 0)),
    ]
    args = [luts_h, g3]
    aliases = {}
    body = _fold_body
    if prev is not None:
        in_specs.append(pl.BlockSpec(memory_space=pltpu.MemorySpace.HBM))
        args.append(prev)
        aliases = {2: 0}
        body = lambda l, g, _p, o: _fold_body(l, g, o)
    return pl.pallas_call(
        body,
        grid=(steps,),
        in_specs=in_specs,
        out_specs=pl.BlockSpec((BATCH, _NT), lambda j: (0, j + base)),
        out_shape=jax.ShapeDtypeStruct((BATCH, NUM_NODES), jnp.float32),
        input_output_aliases=aliases,
    )(*args)


def kernel(x, luts, mapping):
    xt = x.T
    m_t = mapping.T.astype(jnp.int32)
    idx = m_t.reshape(_HROWS)
    g = _sc_gather_half(xt, idx).reshape(N_IN, _HALF, BATCH)
    return _tc_fold_half(luts, g, 0)


# speedup vs baseline: 1.0666x; 1.0666x over previous
import functools

import jax
import jax.numpy as jnp
from jax import lax
from jax.experimental import pallas as pl
from jax.experimental.pallas import tpu as pltpu
from jax.experimental.pallas import tpu_sc as plsc

BATCH = 1024
INPUT_SIZE = 1024
NUM_NODES = 1024
N_IN = 6

_NC, _NS = 2, 16
_NW = _NC * _NS

_HALF = NUM_NODES // 4
_HROWS = N_IN * _HALF
_CHUNK = _HROWS // _NW
_SUB = _CHUNK // 2


def _sc_gather_half(xt, flat_idx):
    @functools.partial(
        pl.kernel,
        mesh=plsc.VectorSubcoreMesh(core_axis_name="c", subcore_axis_name="s"),
        out_type=jax.ShapeDtypeStruct((_HROWS, BATCH), jnp.float32),
        scratch_types=[
            pltpu.VMEM((_SUB,), jnp.int32),
            pltpu.VMEM((_SUB,), jnp.int32),
            pltpu.VMEM((_SUB, BATCH), jnp.float32),
            pltpu.VMEM((_SUB, BATCH), jnp.float32),
            pltpu.SemaphoreType.DMA,
            pltpu.SemaphoreType.DMA,
            pltpu.SemaphoreType.DMA,
        ],
    )
    def body(xt_hbm, idx_hbm, out_hbm, idx0, idx1, rows0, rows1, gsem,
             wsem0, wsem1):
        wid = lax.axis_index("s") * _NC + lax.axis_index("c")
        off = wid * _CHUNK
        pltpu.sync_copy(idx_hbm.at[pl.ds(off, _SUB)], idx0)
        pltpu.sync_copy(idx_hbm.at[pl.ds(off + _SUB, _SUB)], idx1)
        pltpu.async_copy(xt_hbm.at[idx0], rows0, gsem).wait()
        wb0 = pltpu.async_copy(rows0, out_hbm.at[pl.ds(off, _SUB)], wsem0)
        pltpu.async_copy(xt_hbm.at[idx1], rows1, gsem).wait()
        wb1 = pltpu.async_copy(rows1, out_hbm.at[pl.ds(off + _SUB, _SUB)],
                               wsem1)
        wb0.wait()
        wb1.wait()

    return body(xt, flat_idx)


_NT = 128


def _fold_body(luts_ref, g_ref, out_ref):
    g = [g_ref[i] for i in range(N_IN)]

    def t(level, j):
        if level == 1:
            a = luts_ref[:, 2 * j:2 * j + 1]
            b = luts_ref[:, 2 * j + 1:2 * j + 2]
            return a + g[0] * (b - a)
        a = t(level - 1, 2 * j)
        b = t(level - 1, 2 * j + 1)
        return a + g[level - 1] * (b - a)

    out_ref[...] = t(N_IN, 0).T


def _tc_fold_half(luts_h, g3, col0, prev=None):
    steps = _HALF // _NT
    base = col0 // _NT
    in_specs = [
        pl.BlockSpec((_NT, 2 ** N_IN), lambda j: (j, 0)),
        pl.BlockSpec((N_IN, _NT, BATCH), lambda j: (0, j, 0)),
    ]
    args = [luts_h, g3]
    aliases = {}
    body = _fold_body
    if prev is not None:
        in_specs.append(pl.BlockSpec(memory_space=pltpu.MemorySpace.HBM))
        args.append(prev)
        aliases = {2: 0}
        body = lambda l, g, _p, o: _fold_body(l, g, o)
    return pl.pallas_call(
        body,
        grid=(steps,),
        in_specs=in_specs,
        out_specs=pl.BlockSpec((BATCH, _NT), lambda j: (0, j + base)),
        out_shape=jax.ShapeDtypeStruct((BATCH, NUM_NODES), jnp.float32),
        input_output_aliases=aliases,
    )(*args)


def kernel(x, luts, mapping):
    xt = x.T
    m_t = mapping.T.astype(jnp.int32)
    nq = NUM_NODES // _HALF
    gs = [
        _sc_gather_half(
            xt, m_t[:, q * _HALF:(q + 1) * _HALF].reshape(_HROWS)
        ).reshape(N_IN, _HALF, BATCH)
        for q in range(nq)
    ]
    out = _tc_fold_half(luts[:_HALF], gs[0], 0)
    for q in range(1, nq):
        out = _tc_fold_half(luts[q * _HALF:(q + 1) * _HALF], gs[q],
                            q * _HALF, prev=out)
    return out


# speedup vs baseline: 1.1183x; 1.0485x over previous
import functools

import jax
import jax.numpy as jnp
import numpy as np
from jax import lax
from jax.experimental import pallas as pl
from jax.experimental.pallas import tpu as pltpu
from jax.experimental.pallas import tpu_sc as plsc

BATCH = 1024
INPUT_SIZE = 1024
NUM_NODES = 1024
N_IN = 6

_NC, _NS = 2, 16
_NW = _NC * _NS

_HALF = NUM_NODES // 2
_HROWS = N_IN * _HALF
_CHUNK = _HROWS // _NW
_SUB = _CHUNK // 2


def _sc_gather_half(xt, flat_idx):
    @functools.partial(
        pl.kernel,
        mesh=plsc.VectorSubcoreMesh(core_axis_name="c", subcore_axis_name="s"),
        out_type=jax.ShapeDtypeStruct((_HROWS, BATCH), jnp.float32),
        scratch_types=[
            pltpu.VMEM((_SUB,), jnp.int32),
            pltpu.VMEM((_SUB,), jnp.int32),
            pltpu.VMEM((_SUB, BATCH), jnp.float32),
            pltpu.VMEM((_SUB, BATCH), jnp.float32),
            pltpu.SemaphoreType.DMA,
            pltpu.SemaphoreType.DMA,
            pltpu.SemaphoreType.DMA,
        ],
    )
    def body(xt_hbm, idx_hbm, out_hbm, idx0, idx1, rows0, rows1, gsem,
             wsem0, wsem1):
        wid = lax.axis_index("s") * _NC + lax.axis_index("c")
        off = wid * _CHUNK
        pltpu.sync_copy(idx_hbm.at[pl.ds(off, _SUB)], idx0)
        pltpu.sync_copy(idx_hbm.at[pl.ds(off + _SUB, _SUB)], idx1)
        pltpu.async_copy(xt_hbm.at[idx0], rows0, gsem).wait()
        wb0 = pltpu.async_copy(rows0, out_hbm.at[pl.ds(off, _SUB)], wsem0)
        pltpu.async_copy(xt_hbm.at[idx1], rows1, gsem).wait()
        wb1 = pltpu.async_copy(rows1, out_hbm.at[pl.ds(off + _SUB, _SUB)],
                               wsem1)
        wb0.wait()
        wb1.wait()

    return body(xt, flat_idx)


_NT = 128


def _mobius3():
    m = np.zeros((64, 64), dtype=np.float32)
    for j in range(8):
        for e in range(8):
            for s in range(8):
                if (e & s) == e:
                    m[8 * j + e, 8 * j + s] = (
                        -1.0) ** (bin(s).count("1") - bin(e).count("1"))
    return m


_M3 = _mobius3()


def _fold_body(luts_ref, g_ref, m3_ref, out_ref):
    g = [g_ref[i] for i in range(N_IN)]
    cf = jnp.dot(luts_ref[...], m3_ref[...],
                 preferred_element_type=jnp.float32,
                 precision=jax.lax.Precision.HIGHEST)
    p01 = g[0] * g[1]
    p02 = g[0] * g[2]
    p12 = g[1] * g[2]
    p012 = p01 * g[2]

    def t3(j):
        col = lambda s: cf[:, 8 * j + s:8 * j + s + 1]
        return (col(0) + col(1) * g[0] + col(2) * g[1] + col(4) * g[2]
                + col(3) * p01 + col(5) * p02 + col(6) * p12
                + col(7) * p012)

    def t(level, j):
        if level == 3:
            return t3(j)
        a = t(level - 1, 2 * j)
        b = t(level - 1, 2 * j + 1)
        return a + g[level - 1] * (b - a)

    out_ref[...] = t(N_IN, 0).T


def _tc_fold_half(luts_h, g3, col0, prev=None):
    steps = _HALF // _NT
    base = col0 // _NT
    in_specs = [
        pl.BlockSpec((_NT, 2 ** N_IN), lambda j: (j, 0)),
        pl.BlockSpec((N_IN, _NT, BATCH), lambda j: (0, j, 0)),
        pl.BlockSpec((64, 64), lambda j: (0, 0)),
    ]
    args = [luts_h, g3, jnp.asarray(_M3)]
    aliases = {}
    body = _fold_body
    if prev is not None:
        in_specs.append(pl.BlockSpec(memory_space=pltpu.MemorySpace.HBM))
        args.append(prev)
        aliases = {3: 0}
        body = lambda l, g, m, _p, o: _fold_body(l, g, m, o)
    return pl.pallas_call(
        body,
        grid=(steps,),
        in_specs=in_specs,
        out_specs=pl.BlockSpec((BATCH, _NT), lambda j: (0, j + base)),
        out_shape=jax.ShapeDtypeStruct((BATCH, NUM_NODES), jnp.float32),
        input_output_aliases=aliases,
    )(*args)


def kernel(x, luts, mapping):
    xt = x.T
    m_t = mapping.T.astype(jnp.int32)
    idx_a = m_t[:, :_HALF].reshape(_HROWS)
    idx_b = m_t[:, _HALF:].reshape(_HROWS)
    g_a = _sc_gather_half(xt, idx_a).reshape(N_IN, _HALF, BATCH)
    g_b = _sc_gather_half(xt, idx_b).reshape(N_IN, _HALF, BATCH)
    out_a = _tc_fold_half(luts[:_HALF], g_a, 0)
    return _tc_fold_half(luts[_HALF:], g_b, _HALF, prev=out_a)


# speedup vs baseline: 1.1905x; 1.0646x over previous
import functools

import jax
import jax.numpy as jnp
import numpy as np
from jax import lax
from jax.experimental import pallas as pl
from jax.experimental.pallas import tpu as pltpu
from jax.experimental.pallas import tpu_sc as plsc

BATCH = 1024
INPUT_SIZE = 1024
NUM_NODES = 1024
N_IN = 6

_NC, _NS = 2, 16
_NW = _NC * _NS

_HALF = NUM_NODES // 2
_HROWS = N_IN * _HALF
_CHUNK = _HROWS // _NW
_SUB = _CHUNK // 2


def _sc_gather_half(xt, flat_idx):
    @functools.partial(
        pl.kernel,
        mesh=plsc.VectorSubcoreMesh(core_axis_name="c", subcore_axis_name="s"),
        out_type=jax.ShapeDtypeStruct((_HROWS, BATCH // 2), jnp.int32),
        scratch_types=[
            pltpu.VMEM((_SUB,), jnp.int32),
            pltpu.VMEM((_SUB,), jnp.int32),
            pltpu.VMEM((_SUB, BATCH // 2), jnp.int32),
            pltpu.VMEM((_SUB, BATCH // 2), jnp.int32),
            pltpu.SemaphoreType.DMA,
            pltpu.SemaphoreType.DMA,
            pltpu.SemaphoreType.DMA,
        ],
    )
    def body(xt_hbm, idx_hbm, out_hbm, idx0, idx1, rows0, rows1, gsem,
             wsem0, wsem1):
        wid = lax.axis_index("s") * _NC + lax.axis_index("c")
        off = wid * _CHUNK
        pltpu.sync_copy(idx_hbm.at[pl.ds(off, _SUB)], idx0)
        pltpu.sync_copy(idx_hbm.at[pl.ds(off + _SUB, _SUB)], idx1)
        pltpu.async_copy(xt_hbm.at[idx0], rows0, gsem).wait()
        wb0 = pltpu.async_copy(rows0, out_hbm.at[pl.ds(off, _SUB)], wsem0)
        pltpu.async_copy(xt_hbm.at[idx1], rows1, gsem).wait()
        wb1 = pltpu.async_copy(rows1, out_hbm.at[pl.ds(off + _SUB, _SUB)],
                               wsem1)
        wb0.wait()
        wb1.wait()

    return body(xt, flat_idx)


_NT = 128


def _mobius3():
    m = np.zeros((64, 64), dtype=np.float32)
    for j in range(8):
        for e in range(8):
            for s in range(8):
                if (e & s) == e:
                    m[8 * j + e, 8 * j + s] = (
                        -1.0) ** (bin(s).count("1") - bin(e).count("1"))
    return m


_M3 = _mobius3()


def _fold_body(luts_ref, g_ref, m3_ref, out_ref):
    def f16_bits_to_f32(bits):
        f = lax.bitcast_convert_type((bits & 0x7FFF) << 13, jnp.float32)
        return f * jnp.float32(2.0 ** 112)

    def unpack(v32):
        lo = f16_bits_to_f32(v32)
        hi = f16_bits_to_f32((v32 >> 16) & 0xFFFF)
        return jnp.concatenate([lo, hi], axis=1)

    g = [unpack(g_ref[i]) for i in range(N_IN)]
    cf = jnp.dot(luts_ref[...], m3_ref[...],
                 preferred_element_type=jnp.float32,
                 precision=jax.lax.Precision.HIGHEST)
    p01 = g[0] * g[1]
    p02 = g[0] * g[2]
    p12 = g[1] * g[2]
    p012 = p01 * g[2]

    def t3(j):
        col = lambda s: cf[:, 8 * j + s:8 * j + s + 1]
        return (col(0) + col(1) * g[0] + col(2) * g[1] + col(4) * g[2]
                + col(3) * p01 + col(5) * p02 + col(6) * p12
                + col(7) * p012)

    def t(level, j):
        if level == 3:
            return t3(j)
        a = t(level - 1, 2 * j)
        b = t(level - 1, 2 * j + 1)
        return a + g[level - 1] * (b - a)

    out_ref[...] = t(N_IN, 0).T


def _tc_fold_half(luts_h, g3, col0, prev=None):
    steps = _HALF // _NT
    base = col0 // _NT
    in_specs = [
        pl.BlockSpec((_NT, 2 ** N_IN), lambda j: (j, 0)),
        pl.BlockSpec((N_IN, _NT, BATCH // 2), lambda j: (0, j, 0)),
        pl.BlockSpec((64, 64), lambda j: (0, 0)),
    ]
    args = [luts_h, g3, jnp.asarray(_M3)]
    aliases = {}
    body = _fold_body
    if prev is not None:
        in_specs.append(pl.BlockSpec(memory_space=pltpu.MemorySpace.HBM))
        args.append(prev)
        aliases = {3: 0}
        body = lambda l, g, m, _p, o: _fold_body(l, g, m, o)
    return pl.pallas_call(
        body,
        grid=(steps,),
        in_specs=in_specs,
        out_specs=pl.BlockSpec((BATCH, _NT), lambda j: (0, j + base)),
        out_shape=jax.ShapeDtypeStruct((BATCH, NUM_NODES), jnp.float32),
        input_output_aliases=aliases,
    )(*args)


def kernel(x, luts, mapping):
    xh = lax.bitcast_convert_type(x.T.astype(jnp.float16),
                                  jnp.uint16).astype(jnp.uint32)
    xt = (xh[:, :BATCH // 2] | (xh[:, BATCH // 2:] << 16)).astype(jnp.int32)
    m_t = mapping.T.astype(jnp.int32)
    idx_a = m_t[:, :_HALF].reshape(_HROWS)
    idx_b = m_t[:, _HALF:].reshape(_HROWS)
    g_a = _sc_gather_half(xt, idx_a).reshape(N_IN, _HALF, BATCH // 2)
    g_b = _sc_gather_half(xt, idx_b).reshape(N_IN, _HALF, BATCH // 2)
    out_a = _tc_fold_half(luts[:_HALF], g_a, 0)
    return _tc_fold_half(luts[_HALF:], g_b, _HALF, prev=out_a)


# speedup vs baseline: 1.1932x; 1.0023x over previous
import functools

import jax
import jax.numpy as jnp
import numpy as np
from jax import lax
from jax.experimental import pallas as pl
from jax.experimental.pallas import tpu as pltpu
from jax.experimental.pallas import tpu_sc as plsc

BATCH = 1024
INPUT_SIZE = 1024
NUM_NODES = 1024
N_IN = 6

_NC, _NS = 2, 16
_NW = _NC * _NS

_HALF = NUM_NODES // 2
_HROWS = N_IN * _HALF
_CHUNK = _HROWS // _NW
_SUB = _CHUNK // 2


def _sc_gather_half(xt, flat_idx):
    @functools.partial(
        pl.kernel,
        mesh=plsc.VectorSubcoreMesh(core_axis_name="c", subcore_axis_name="s"),
        out_type=jax.ShapeDtypeStruct((_HROWS, BATCH // 2), jnp.int32),
        scratch_types=[
            pltpu.VMEM((_SUB,), jnp.int32),
            pltpu.VMEM((_SUB,), jnp.int32),
            pltpu.VMEM((_SUB, BATCH // 2), jnp.int32),
            pltpu.VMEM((_SUB, BATCH // 2), jnp.int32),
            pltpu.SemaphoreType.DMA,
            pltpu.SemaphoreType.DMA,
            pltpu.SemaphoreType.DMA,
        ],
    )
    def body(xt_hbm, idx_hbm, out_hbm, idx0, idx1, rows0, rows1, gsem,
             wsem0, wsem1):
        wid = lax.axis_index("s") * _NC + lax.axis_index("c")
        off = wid * _CHUNK
        pltpu.sync_copy(idx_hbm.at[pl.ds(off, _SUB)], idx0)
        pltpu.sync_copy(idx_hbm.at[pl.ds(off + _SUB, _SUB)], idx1)
        pltpu.async_copy(xt_hbm.at[idx0], rows0, gsem).wait()
        wb0 = pltpu.async_copy(rows0, out_hbm.at[pl.ds(off, _SUB)], wsem0)
        pltpu.async_copy(xt_hbm.at[idx1], rows1, gsem).wait()
        wb1 = pltpu.async_copy(rows1, out_hbm.at[pl.ds(off + _SUB, _SUB)],
                               wsem1)
        wb0.wait()
        wb1.wait()

    return body(xt, flat_idx)


_NT = 128


def _mobius3():
    m = np.zeros((64, 64), dtype=np.float32)
    for j in range(8):
        for e in range(8):
            for s in range(8):
                if (e & s) == e:
                    m[8 * j + e, 8 * j + s] = (
                        -1.0) ** (bin(s).count("1") - bin(e).count("1"))
    return m


_M3 = _mobius3()


def _fold_body(luts_ref, g_ref, m3_ref, out_ref):
    def f16_bits_to_f32(bits):
        f = lax.bitcast_convert_type((bits & 0x7FFF) << 13, jnp.float32)
        return f * jnp.float32(2.0 ** 112)

    def unpack(v32):
        lo = f16_bits_to_f32(v32)
        hi = f16_bits_to_f32((v32 >> 16) & 0xFFFF)
        return jnp.concatenate([lo, hi], axis=1)

    g = [unpack(g_ref[i]) for i in range(N_IN)]
    cf = jnp.dot(luts_ref[...], m3_ref[...],
                 preferred_element_type=jnp.float32,
                 precision=jax.lax.Precision.HIGHEST)
    p01 = g[0] * g[1]
    p02 = g[0] * g[2]
    p12 = g[1] * g[2]
    p012 = p01 * g[2]

    def t3(j):
        col = lambda s: cf[:, 8 * j + s:8 * j + s + 1]
        return (col(0) + col(1) * g[0] + col(2) * g[1] + col(4) * g[2]
                + col(3) * p01 + col(5) * p02 + col(6) * p12
                + col(7) * p012)

    def t(level, j):
        if level == 3:
            return t3(j)
        a = t(level - 1, 2 * j)
        b = t(level - 1, 2 * j + 1)
        return a + g[level - 1] * (b - a)

    out_ref[...] = t(N_IN, 0).T


def _tc_fold_half(luts_h, g3, col0, prev=None):
    steps = _HALF // _NT
    base = col0 // _NT
    in_specs = [
        pl.BlockSpec((_NT, 2 ** N_IN), lambda j: (j, 0)),
        pl.BlockSpec((N_IN, _NT, BATCH // 2), lambda j: (0, j, 0)),
        pl.BlockSpec((64, 64), lambda j: (0, 0)),
    ]
    args = [luts_h, g3, jnp.asarray(_M3)]
    aliases = {}
    body = _fold_body
    if prev is not None:
        in_specs.append(pl.BlockSpec(memory_space=pltpu.MemorySpace.HBM))
        args.append(prev)
        aliases = {3: 0}
        body = lambda l, g, m, _p, o: _fold_body(l, g, m, o)
    return pl.pallas_call(
        body,
        grid=(steps,),
        in_specs=in_specs,
        out_specs=pl.BlockSpec((BATCH, _NT), lambda j: (0, j + base)),
        out_shape=jax.ShapeDtypeStruct((BATCH, NUM_NODES), jnp.float32),
        input_output_aliases=aliases,
    )(*args)


def kernel(x, luts, mapping):
    xh = lax.bitcast_convert_type(x.astype(jnp.float16),
                                  jnp.uint16).astype(jnp.uint32)
    xt = (xh[:BATCH // 2, :] | (xh[BATCH // 2:, :] << 16)).T.astype(jnp.int32)
    m_t = mapping.T.astype(jnp.int32)
    idx_a = m_t[:, :_HALF].reshape(_HROWS)
    idx_b = m_t[:, _HALF:].reshape(_HROWS)
    g_a = _sc_gather_half(xt, idx_a).reshape(N_IN, _HALF, BATCH // 2)
    g_b = _sc_gather_half(xt, idx_b).reshape(N_IN, _HALF, BATCH // 2)
    out_a = _tc_fold_half(luts[:_HALF], g_a, 0)
    return _tc_fold_half(luts[_HALF:], g_b, _HALF, prev=out_a)


# speedup vs baseline: 1.2160x; 1.0191x over previous
import functools

import jax
import jax.numpy as jnp
import numpy as np
from jax import lax
from jax.experimental import pallas as pl
from jax.experimental.pallas import tpu as pltpu
from jax.experimental.pallas import tpu_sc as plsc

BATCH = 1024
INPUT_SIZE = 1024
NUM_NODES = 1024
N_IN = 6

_NC, _NS = 2, 16
_NW = _NC * _NS

_HALF = NUM_NODES // 2
_HROWS = N_IN * _HALF
_CHUNK = _HROWS // _NW
_SUB = _CHUNK // 2


def _sc_gather_half(xt, flat_idx):
    @functools.partial(
        pl.kernel,
        mesh=plsc.VectorSubcoreMesh(core_axis_name="c", subcore_axis_name="s"),
        out_type=jax.ShapeDtypeStruct((_HROWS, BATCH // 2), jnp.int32),
        scratch_types=[
            pltpu.VMEM((_CHUNK,), jnp.int32),
            pltpu.VMEM((_CHUNK, BATCH // 2), jnp.int32),
            pltpu.SemaphoreType.DMA,
            pltpu.SemaphoreType.DMA,
        ],
    )
    def body(xt_hbm, idx_hbm, out_hbm, idx0, rows0, gsem, wsem0):
        wid = lax.axis_index("s") * _NC + lax.axis_index("c")
        off = wid * _CHUNK
        pltpu.sync_copy(idx_hbm.at[pl.ds(off, _CHUNK)], idx0)
        pltpu.async_copy(xt_hbm.at[idx0], rows0, gsem).wait()
        pltpu.async_copy(rows0, out_hbm.at[pl.ds(off, _CHUNK)], wsem0).wait()

    return body(xt, flat_idx)


_NT = 128


def _mobius3():
    m = np.zeros((64, 64), dtype=np.float32)
    for j in range(8):
        for e in range(8):
            for s in range(8):
                if (e & s) == e:
                    m[8 * j + e, 8 * j + s] = (
                        -1.0) ** (bin(s).count("1") - bin(e).count("1"))
    return m


_M3 = _mobius3()


def _fold_body(luts_ref, g_ref, m3_ref, out_ref):
    def f16_bits_to_f32(bits):
        f = lax.bitcast_convert_type((bits & 0x7FFF) << 13, jnp.float32)
        return f * jnp.float32(2.0 ** 112)

    def unpack(v32):
        lo = f16_bits_to_f32(v32)
        hi = f16_bits_to_f32((v32 >> 16) & 0xFFFF)
        return jnp.concatenate([lo, hi], axis=1)

    g = [unpack(g_ref[i]) for i in range(N_IN)]
    cf = jnp.dot(luts_ref[...], m3_ref[...],
                 preferred_element_type=jnp.float32,
                 precision=jax.lax.Precision.HIGHEST)
    p01 = g[0] * g[1]
    p02 = g[0] * g[2]
    p12 = g[1] * g[2]
    p012 = p01 * g[2]

    def t3(j):
        col = lambda s: cf[:, 8 * j + s:8 * j + s + 1]
        return (col(0) + col(1) * g[0] + col(2) * g[1] + col(4) * g[2]
                + col(3) * p01 + col(5) * p02 + col(6) * p12
                + col(7) * p012)

    def t(level, j):
        if level == 3:
            return t3(j)
        a = t(level - 1, 2 * j)
        b = t(level - 1, 2 * j + 1)
        return a + g[level - 1] * (b - a)

    out_ref[...] = t(N_IN, 0).T


def _tc_fold_half(luts_h, g3, col0, prev=None):
    steps = _HALF // _NT
    base = col0 // _NT
    in_specs = [
        pl.BlockSpec((_NT, 2 ** N_IN), lambda j: (j, 0)),
        pl.BlockSpec((N_IN, _NT, BATCH // 2), lambda j: (0, j, 0)),
        pl.BlockSpec((64, 64), lambda j: (0, 0)),
    ]
    args = [luts_h, g3, jnp.asarray(_M3)]
    aliases = {}
    body = _fold_body
    if prev is not None:
        in_specs.append(pl.BlockSpec(memory_space=pltpu.MemorySpace.HBM))
        args.append(prev)
        aliases = {3: 0}
        body = lambda l, g, m, _p, o: _fold_body(l, g, m, o)
    return pl.pallas_call(
        body,
        grid=(steps,),
        in_specs=in_specs,
        out_specs=pl.BlockSpec((BATCH, _NT), lambda j: (0, j + base)),
        out_shape=jax.ShapeDtypeStruct((BATCH, NUM_NODES), jnp.float32),
        input_output_aliases=aliases,
    )(*args)


def kernel(x, luts, mapping):
    xh = lax.bitcast_convert_type(x.astype(jnp.float16),
                                  jnp.uint16).astype(jnp.uint32)
    xt = (xh[:BATCH // 2, :] | (xh[BATCH // 2:, :] << 16)).T.astype(jnp.int32)
    m_t = mapping.T.astype(jnp.int32)
    idx_a = m_t[:, :_HALF].reshape(_HROWS)
    idx_b = m_t[:, _HALF:].reshape(_HROWS)
    g_a = _sc_gather_half(xt, idx_a).reshape(N_IN, _HALF, BATCH // 2)
    g_b = _sc_gather_half(xt, idx_b).reshape(N_IN, _HALF, BATCH // 2)
    out_a = _tc_fold_half(luts[:_HALF], g_a, 0)
    return _tc_fold_half(luts[_HALF:], g_b, _HALF, prev=out_a)
